# baseline (speedup 1.0000x reference)
import functools

import jax
import jax.numpy as jnp
from jax import lax
from jax.experimental import pallas as pl
from jax.experimental.pallas import tpu as pltpu
from jax.experimental.pallas import tpu_sc as plsc

NC = 2
NS = 16
NW = NC * NS
L = 16


def _phase_a(nodes, src, dst, features):
    B = nodes.shape[0]
    E = src.shape[0]
    N, D = features.shape
    E_PER = E // NW
    B_PER = B // NW
    SLOTS = B + 8 * NS
    K = 128
    NCH = (E_PER + K - 1) // K
    CAP = NCH * K + K
    RPT = SLOTS // NS
    NPAD = ((N + 127) // 128) * 128

    mesh = plsc.VectorSubcoreMesh(core_axis_name="c", subcore_axis_name="s")

    @functools.partial(
        pl.kernel,
        out_type=[
            jax.ShapeDtypeStruct((SLOTS, D), jnp.float32),
            jax.ShapeDtypeStruct((SLOTS, D), jnp.float32),
            jax.ShapeDtypeStruct((SLOTS,), jnp.float32),
            jax.ShapeDtypeStruct((SLOTS,), jnp.float32),
            jax.ShapeDtypeStruct((B,), jnp.int32),
        ],
        mesh=mesh,
        compiler_params=pltpu.CompilerParams(needs_layout_passes=False,
                                             use_tc_tiling_on_sc=False),
        scratch_types=[
            pltpu.VMEM((NPAD,), jnp.int32),
            pltpu.VMEM((B,), jnp.int32),
            pltpu.VMEM((E_PER,), jnp.int32),
            pltpu.VMEM((E_PER,), jnp.int32),
            pltpu.VMEM((CAP,), jnp.int32),
            pltpu.VMEM((CAP,), jnp.int32),
            pltpu.VMEM((3, K, D), jnp.float32),
            pltpu.VMEM((K, L), jnp.float32),
            pltpu.VMEM((B_PER,), jnp.int32),
            pltpu.VMEM((RPT, L), jnp.float32),
            pltpu.VMEM((RPT + L, L), jnp.float32),
            pltpu.VMEM((((RPT + L - 1) // L) * L,), jnp.float32),
            pltpu.VMEM_SHARED((SLOTS, D), jnp.float32),
            pltpu.VMEM_SHARED((SLOTS, L), jnp.float32),
            pltpu.SemaphoreType.DMA,
            pltpu.SemaphoreType.DMA,
            pltpu.SemaphoreType.DMA,
            pltpu.SemaphoreType.DMA,
        ],
    )
    def k(nodes_h, src_h, dst_h, feat_h, acc0_h, acc1_h, cnt0_h, cnt1_h, g_h,
          tbl, nod, srcv, dstv, lsrc, lslot, gbuf, ones, gch, zc,
          cbandv, cband1, acc, cnt, sem, sg0, sg1, sg2):
        c = lax.axis_index("c")
        s = lax.axis_index("s")
        wid = s * NC + c

        pltpu.sync_copy(nodes_h, nod)

        neg1 = jnp.full((L,), -1, jnp.int32)
        def init_t(i, _):
            tbl[pl.ds(i * L, L)] = neg1
            return 0
        lax.fori_loop(0, NPAD // L, init_t, 0, unroll=8)
        iota = lax.iota(jnp.int32, L)
        def fill_t(i, _):
            plsc.store_scatter(tbl, [nod[pl.ds(i * L, L)]], iota + i * L)
            return 0
        lax.fori_loop(0, B // L, fill_t, 0, unroll=8)

        bb = wid * B_PER
        def fill_g(i, _):
            gch[pl.ds(i * L, L)] = plsc.load_gather(tbl, [nod[pl.ds(bb + i * L, L)]])
            return 0
        lax.fori_loop(0, B_PER // L, fill_g, 0)
        pltpu.sync_copy(gch, g_h.at[pl.ds(bb, B_PER)])

        zero = jnp.zeros((L,), jnp.float32)
        one = jnp.ones((L,), jnp.float32)
        def zfill(i, _):
            for q in range(D // L):
                gbuf[0, i, pl.ds(q * L, L)] = zero
            zc[i] = zero
            return 0
        lax.fori_loop(0, RPT, zfill, 0, unroll=4)
        def ofill(i, _):
            ones[i] = one
            return 0
        lax.fori_loop(0, K, ofill, 0, unroll=8)
        r0 = s * RPT
        pltpu.sync_copy(gbuf.at[0, pl.ds(0, RPT)], acc.at[pl.ds(r0, RPT)])
        pltpu.sync_copy(zc, cnt.at[pl.ds(r0, RPT)])
        plsc.subcore_barrier()

        eb = wid * E_PER
        pltpu.sync_copy(src_h.at[pl.ds(eb, E_PER)], srcv)
        pltpu.sync_copy(dst_h.at[pl.ds(eb, E_PER)], dstv)

        def filt(j, m):
            d = dstv[pl.ds(j * L, L)]
            sl = plsc.load_gather(tbl, [d])
            msk = sl >= 0
            cum = plsc.cumsum(msk.astype(jnp.int32))
            pos = m + cum - 1
            plsc.store_scatter(lsrc, [pos], srcv[pl.ds(j * L, L)], mask=msk)
            plsc.store_scatter(lslot, [pos], sl, mask=msk)
            return m + jnp.max(cum)
        m = lax.fori_loop(0, E_PER // L, filt, jnp.int32(0), unroll=4)

        zero_i = jnp.zeros((L,), jnp.int32)
        dummy = jnp.full((L,), B, jnp.int32)
        for p in range(K // L):
            pidx = m + iota + p * L
            plsc.store_scatter(lsrc, [pidx], zero_i)
            plsc.store_scatter(lslot, [pidx], dummy)

        nch = (m + (K - 1)) // K
        sgs = [sg0, sg1, sg2]
        NB = 3
        def issue(o, b):
            pltpu.async_copy(feat_h.at[lsrc.at[pl.ds(o, K)]],
                             gbuf.at[b], sgs[b])

        def drain(o, b):
            pltpu.make_async_copy(feat_h.at[lsrc.at[pl.ds(o, K)]],
                                  gbuf.at[b], sgs[b]).wait()

        for b in range(NB):
            @pl.when(b < nch)
            def _(b=b):
                issue(b * K, b)
        def ring(tt, _):
            for b in range(NB):
                t = tt * NB + b
                @pl.when(t < nch)
                def _(b=b, t=t):
                    o = pl.multiple_of(t * K, K)
                    drain(o, b)
                    idx = lslot.at[pl.ds(o, K)]
                    pltpu.sync_copy(gbuf.at[b], acc.at[idx], add=True)
                    pltpu.sync_copy(ones, cnt.at[idx], add=True)
                    tn = t + NB
                    @pl.when(tn < nch)
                    def _():
                        issue(pl.multiple_of(tn * K, K), b)
            return 0
        lax.fori_loop(0, (nch + NB - 1) // NB, ring, 0)
        plsc.subcore_barrier()

        pltpu.sync_copy(cnt.at[pl.ds(r0, RPT)], cbandv.at[pl.ds(0, RPT)])
        zidx = jnp.zeros((L,), jnp.int32)
        for t in range((RPT + L - 1) // L):
            rows = iota + t * L
            cband1[pl.ds(t * L, L)] = plsc.load_gather(cbandv, [rows, zidx])

        @pl.when(c == 0)
        def _():
            pltpu.sync_copy(acc.at[pl.ds(r0, RPT)], acc0_h.at[pl.ds(r0, RPT)])
            pltpu.sync_copy(cband1.at[pl.ds(0, RPT)], cnt0_h.at[pl.ds(r0, RPT)])
        @pl.when(c == 1)
        def _():
            pltpu.sync_copy(acc.at[pl.ds(r0, RPT)], acc1_h.at[pl.ds(r0, RPT)])
            pltpu.sync_copy(cband1.at[pl.ds(0, RPT)], cnt1_h.at[pl.ds(r0, RPT)])

    return k(nodes, src, dst, features)


def _phase_b(nodes, g, acc0, acc1, cnt0, cnt1, features):
    B = nodes.shape[0]
    N, D = features.shape
    B_PER = B // NW
    SLOTS = cnt0.shape[0]

    mesh = plsc.VectorSubcoreMesh(core_axis_name="c", subcore_axis_name="s")

    @functools.partial(
        pl.kernel,
        out_type=jax.ShapeDtypeStruct((2, B, D), jnp.float32),
        mesh=mesh,
        compiler_params=pltpu.CompilerParams(needs_layout_passes=False),
        scratch_types=[
            pltpu.VMEM((B_PER,), jnp.int32),
            pltpu.VMEM((B_PER,), jnp.int32),
            pltpu.VMEM((B_PER, D), jnp.float32),
            pltpu.VMEM((B_PER, D), jnp.float32),
            pltpu.VMEM((B_PER, D), jnp.float32),
            pltpu.VMEM((B_PER, D), jnp.float32),
            pltpu.VMEM((SLOTS,), jnp.float32),
            pltpu.VMEM((SLOTS,), jnp.float32),
            pltpu.VMEM((B_PER + L,), jnp.float32),
            pltpu.SemaphoreType.DMA,
        ],
    )
    def k(nodes_h, g_h, acc0_h, acc1_h, cnt0_h, cnt1_h, feat_h, out_h,
          nv, gv, sb, a0, a1, nb, c0, c1, rv, sem):
        c = lax.axis_index("c")
        s = lax.axis_index("s")
        wid = s * NC + c
        base = wid * B_PER
        pltpu.sync_copy(nodes_h.at[pl.ds(base, B_PER)], nv)
        pltpu.sync_copy(g_h.at[pl.ds(base, B_PER)], gv)
        cp1 = pltpu.async_copy(feat_h.at[nv], sb, sem)
        cp2 = pltpu.async_copy(acc0_h.at[gv], a0, sem)
        cp3 = pltpu.async_copy(acc1_h.at[gv], a1, sem)
        cp4 = pltpu.async_copy(cnt0_h, c0, sem)
        cp5 = pltpu.async_copy(cnt1_h, c1, sem)
        cp1.wait(); cp2.wait(); cp3.wait(); cp4.wait(); cp5.wait()
        for t in range(B_PER // L):
            gvec = gv[pl.ds(t * L, L)]
            cv = plsc.load_gather(c0, [gvec]) + plsc.load_gather(c1, [gvec])
            rv[pl.ds(t * L, L)] = 1.0 / jnp.maximum(cv, 1.0)
        def row(r, _):
            scale = rv[pl.ds(r, L)][0]
            for q in range(D // L):
                nb[r, pl.ds(q * L, L)] = (
                    a0[r, pl.ds(q * L, L)] + a1[r, pl.ds(q * L, L)]) * scale
            return 0
        lax.fori_loop(0, B_PER, row, 0)
        pltpu.sync_copy(sb, out_h.at[0, pl.ds(base, B_PER)])
        pltpu.sync_copy(nb, out_h.at[1, pl.ds(base, B_PER)])

    return k(nodes, g, acc0, acc1, cnt0, cnt1, features)


def _phase_c(comb, W_enc, weight):
    B, D = comb.shape[1], comb.shape[2]

    def body(cb, we, wc, ob):
        sfeat = cb[0]
        nfeat = cb[1]
        w1 = we[:, :D]
        w2 = we[:, D:]
        e = lax.dot_general(sfeat, w1, (((1,), (1,)), ((), ())),
                            preferred_element_type=jnp.float32)
        e = e + lax.dot_general(nfeat, w2, (((1,), (1,)), ((), ())),
                                preferred_element_type=jnp.float32)
        e = jnp.maximum(e, 0.0)
        ob[...] = lax.dot_general(e, wc[...], (((1,), (1,)), ((), ())),
                                  preferred_element_type=jnp.float32)

    return pl.pallas_call(
        body,
        out_shape=jax.ShapeDtypeStruct((B, weight.shape[0]), jnp.float32),
    )(comb, W_enc, weight)


def kernel(nodes, edge_index, features, W_enc, weight):
    src = edge_index[0]
    dst = edge_index[1]
    acc0, acc1, cnt0, cnt1, g = _phase_a(nodes, src, dst, features)
    comb = _phase_b(nodes, g, acc0, acc1, cnt0, cnt1, features)
    return _phase_c(comb, W_enc, weight)


# speedup vs baseline: 1.0373x; 1.0373x over previous
import functools

import jax
import jax.numpy as jnp
from jax import lax
from jax.experimental import pallas as pl
from jax.experimental.pallas import tpu as pltpu
from jax.experimental.pallas import tpu_sc as plsc

NC = 2
NS = 16
NW = NC * NS
L = 16


def _phase_a(nodes, src, dst, features):
    B = nodes.shape[0]
    E = src.shape[0]
    N, D = features.shape
    E_PER = E // NW
    B_PER = B // NW
    SLOTS = B + 8 * NS
    K = 128
    NCH = (E_PER + K - 1) // K
    CAP = NCH * K + K
    RPT = SLOTS // NS
    NPAD = ((N + 127) // 128) * 128

    mesh = plsc.VectorSubcoreMesh(core_axis_name="c", subcore_axis_name="s")

    @functools.partial(
        pl.kernel,
        out_type=[
            jax.ShapeDtypeStruct((SLOTS, D), jnp.float32),
            jax.ShapeDtypeStruct((SLOTS, D), jnp.float32),
            jax.ShapeDtypeStruct((SLOTS,), jnp.float32),
            jax.ShapeDtypeStruct((SLOTS,), jnp.float32),
            jax.ShapeDtypeStruct((B,), jnp.int32),
            jax.ShapeDtypeStruct((B, D), jnp.float32),
        ],
        mesh=mesh,
        compiler_params=pltpu.CompilerParams(needs_layout_passes=False,
                                             use_tc_tiling_on_sc=False),
        scratch_types=[
            pltpu.VMEM((NPAD,), jnp.int32),
            pltpu.VMEM((B,), jnp.int32),
            pltpu.VMEM((E_PER,), jnp.int32),
            pltpu.VMEM((E_PER,), jnp.int32),
            pltpu.VMEM((CAP,), jnp.int32),
            pltpu.VMEM((CAP,), jnp.int32),
            pltpu.VMEM((3, K, D), jnp.float32),
            pltpu.VMEM((K, L), jnp.float32),
            pltpu.VMEM((B_PER,), jnp.int32),
            pltpu.VMEM((B_PER, D), jnp.float32),
            pltpu.VMEM((RPT, L), jnp.float32),
            pltpu.VMEM((RPT + L, L), jnp.float32),
            pltpu.VMEM((((RPT + L - 1) // L) * L,), jnp.float32),
            pltpu.VMEM_SHARED((SLOTS, D), jnp.float32),
            pltpu.VMEM_SHARED((SLOTS, L), jnp.float32),
            pltpu.SemaphoreType.DMA,
            pltpu.SemaphoreType.DMA,
            pltpu.SemaphoreType.DMA,
            pltpu.SemaphoreType.DMA,
        ],
    )
    def k(nodes_h, src_h, dst_h, feat_h, acc0_h, acc1_h, cnt0_h, cnt1_h, g_h,
          self_h, tbl, nod, srcv, dstv, lsrc, lslot, gbuf, ones, gch, sbuf,
          zc, cbandv, cband1, acc, cnt, sem, sg0, sg1, sg2):
        c = lax.axis_index("c")
        s = lax.axis_index("s")
        wid = s * NC + c

        pltpu.sync_copy(nodes_h, nod)

        neg1 = jnp.full((L,), -1, jnp.int32)
        def init_t(i, _):
            tbl[pl.ds(i * L, L)] = neg1
            return 0
        lax.fori_loop(0, NPAD // L, init_t, 0, unroll=8)
        iota = lax.iota(jnp.int32, L)
        def fill_t(i, _):
            plsc.store_scatter(tbl, [nod[pl.ds(i * L, L)]], iota + i * L)
            return 0
        lax.fori_loop(0, B // L, fill_t, 0, unroll=8)

        bb = wid * B_PER
        def fill_g(i, _):
            gch[pl.ds(i * L, L)] = plsc.load_gather(tbl, [nod[pl.ds(bb + i * L, L)]])
            return 0
        lax.fori_loop(0, B_PER // L, fill_g, 0)
        pltpu.sync_copy(gch, g_h.at[pl.ds(bb, B_PER)])

        pltpu.async_copy(feat_h.at[nod.at[pl.ds(bb, B_PER)]], sbuf, sem).wait()
        pltpu.sync_copy(sbuf, self_h.at[pl.ds(bb, B_PER)])

        zero = jnp.zeros((L,), jnp.float32)
        one = jnp.ones((L,), jnp.float32)
        def zfill(i, _):
            for q in range(D // L):
                gbuf[0, i, pl.ds(q * L, L)] = zero
            zc[i] = zero
            return 0
        lax.fori_loop(0, RPT, zfill, 0, unroll=4)
        def ofill(i, _):
            ones[i] = one
            return 0
        lax.fori_loop(0, K, ofill, 0, unroll=8)
        r0 = s * RPT
        pltpu.sync_copy(gbuf.at[0, pl.ds(0, RPT)], acc.at[pl.ds(r0, RPT)])
        pltpu.sync_copy(zc, cnt.at[pl.ds(r0, RPT)])
        plsc.subcore_barrier()

        eb = wid * E_PER
        pltpu.sync_copy(src_h.at[pl.ds(eb, E_PER)], srcv)
        pltpu.sync_copy(dst_h.at[pl.ds(eb, E_PER)], dstv)

        def filt(j, m):
            d = dstv[pl.ds(j * L, L)]
            sl = plsc.load_gather(tbl, [d])
            msk = sl >= 0
            cum = plsc.cumsum(msk.astype(jnp.int32))
            pos = m + cum - 1
            plsc.store_scatter(lsrc, [pos], srcv[pl.ds(j * L, L)], mask=msk)
            plsc.store_scatter(lslot, [pos], sl, mask=msk)
            return m + jnp.max(cum)
        m = lax.fori_loop(0, E_PER // L, filt, jnp.int32(0), unroll=4)

        zero_i = jnp.zeros((L,), jnp.int32)
        dummy = jnp.full((L,), B, jnp.int32)
        for p in range(K // L):
            pidx = m + iota + p * L
            plsc.store_scatter(lsrc, [pidx], zero_i)
            plsc.store_scatter(lslot, [pidx], dummy)

        nch = (m + (K - 1)) // K
        sgs = [sg0, sg1, sg2]
        NB = 3
        def issue(o, b):
            pltpu.async_copy(feat_h.at[lsrc.at[pl.ds(o, K)]],
                             gbuf.at[b], sgs[b])

        def drain(o, b):
            pltpu.make_async_copy(feat_h.at[lsrc.at[pl.ds(o, K)]],
                                  gbuf.at[b], sgs[b]).wait()

        for b in range(NB):
            @pl.when(b < nch)
            def _(b=b):
                issue(b * K, b)
        def ring(tt, _):
            for b in range(NB):
                t = tt * NB + b
                @pl.when(t < nch)
                def _(b=b, t=t):
                    o = pl.multiple_of(t * K, K)
                    drain(o, b)
                    idx = lslot.at[pl.ds(o, K)]
                    pltpu.sync_copy(gbuf.at[b], acc.at[idx], add=True)
                    pltpu.sync_copy(ones, cnt.at[idx], add=True)
                    tn = t + NB
                    @pl.when(tn < nch)
                    def _():
                        issue(pl.multiple_of(tn * K, K), b)
            return 0
        lax.fori_loop(0, (nch + NB - 1) // NB, ring, 0)
        plsc.subcore_barrier()

        pltpu.sync_copy(cnt.at[pl.ds(r0, RPT)], cbandv.at[pl.ds(0, RPT)])
        zidx = jnp.zeros((L,), jnp.int32)
        for t in range((RPT + L - 1) // L):
            rows = iota + t * L
            cband1[pl.ds(t * L, L)] = plsc.load_gather(cbandv, [rows, zidx])

        @pl.when(c == 0)
        def _():
            pltpu.sync_copy(acc.at[pl.ds(r0, RPT)], acc0_h.at[pl.ds(r0, RPT)])
            pltpu.sync_copy(cband1.at[pl.ds(0, RPT)], cnt0_h.at[pl.ds(r0, RPT)])
        @pl.when(c == 1)
        def _():
            pltpu.sync_copy(acc.at[pl.ds(r0, RPT)], acc1_h.at[pl.ds(r0, RPT)])
            pltpu.sync_copy(cband1.at[pl.ds(0, RPT)], cnt1_h.at[pl.ds(r0, RPT)])

    return k(nodes, src, dst, features)


def _phase_c(selfb, g, acc0, acc1, cnt0, cnt1, W_enc, weight):
    B, D = selfb.shape
    SLOTS = acc0.shape[0]

    def body(sref, gref, a0, a1, c0, c1, we, wc, ob):
        gv = gref[...]
        onehot = (gv[:, None] ==
                  lax.broadcasted_iota(jnp.int32, (B, SLOTS), 1)
                  ).astype(jnp.float32)
        macc = jnp.concatenate(
            [a0[...] + a1[...], (c0[...] + c1[...])[:, None]], axis=1)
        p = lax.dot_general(onehot, macc, (((1,), (0,)), ((), ())),
                            preferred_element_type=jnp.float32)
        neigh = p[:, :D] / jnp.maximum(p[:, D:D + 1], 1.0)
        w1 = we[:, :D]
        w2 = we[:, D:]
        e = lax.dot_general(sref[...], w1, (((1,), (1,)), ((), ())),
                            preferred_element_type=jnp.float32)
        e = e + lax.dot_general(neigh, w2, (((1,), (1,)), ((), ())),
                                preferred_element_type=jnp.float32)
        e = jnp.maximum(e, 0.0)
        ob[...] = lax.dot_general(e, wc[...], (((1,), (1,)), ((), ())),
                                  preferred_element_type=jnp.float32)

    return pl.pallas_call(
        body,
        out_shape=jax.ShapeDtypeStruct((B, weight.shape[0]), jnp.float32),
    )(selfb, g, acc0, acc1, cnt0, cnt1, W_enc, weight)


def kernel(nodes, edge_index, features, W_enc, weight):
    src = edge_index[0]
    dst = edge_index[1]
    acc0, acc1, cnt0, cnt1, g, selfb = _phase_a(nodes, src, dst, features)
    return _phase_c(selfb, g, acc0, acc1, cnt0, cnt1, W_enc, weight)


# speedup vs baseline: 1.0382x; 1.0008x over previous
import functools

import jax
import jax.numpy as jnp
from jax import lax
from jax.experimental import pallas as pl
from jax.experimental.pallas import tpu as pltpu
from jax.experimental.pallas import tpu_sc as plsc

NC = 2
NS = 16
NW = NC * NS
L = 16


def _phase_a(nodes, src, dst, features):
    B = nodes.shape[0]
    E = src.shape[0]
    N, D = features.shape
    E_PER = E // NW
    B_PER = B // NW
    SLOTS = B + 8 * NS
    K = 128
    NCH = (E_PER + K - 1) // K
    CAP = NCH * K + K
    RPT = SLOTS // NS
    NPAD = ((N + 127) // 128) * 128

    mesh = plsc.VectorSubcoreMesh(core_axis_name="c", subcore_axis_name="s")

    @functools.partial(
        pl.kernel,
        out_type=[
            jax.ShapeDtypeStruct((SLOTS, D), jnp.float32),
            jax.ShapeDtypeStruct((SLOTS, D), jnp.float32),
            jax.ShapeDtypeStruct((SLOTS,), jnp.float32),
            jax.ShapeDtypeStruct((SLOTS,), jnp.float32),
            jax.ShapeDtypeStruct((B,), jnp.int32),
            jax.ShapeDtypeStruct((B, D), jnp.float32),
        ],
        mesh=mesh,
        compiler_params=pltpu.CompilerParams(needs_layout_passes=False,
                                             use_tc_tiling_on_sc=False),
        scratch_types=[
            pltpu.VMEM((NPAD,), jnp.int32),
            pltpu.VMEM((B,), jnp.int32),
            pltpu.VMEM((E_PER,), jnp.int32),
            pltpu.VMEM((E_PER,), jnp.int32),
            pltpu.VMEM((CAP,), jnp.int32),
            pltpu.VMEM((CAP,), jnp.int32),
            pltpu.VMEM((3, K, D), jnp.float32),
            pltpu.VMEM((K, L), jnp.float32),
            pltpu.VMEM((B_PER,), jnp.int32),
            pltpu.VMEM((B_PER, D), jnp.float32),
            pltpu.VMEM((RPT, D), jnp.float32),
            pltpu.VMEM((RPT, L), jnp.float32),
            pltpu.VMEM((RPT + L, L), jnp.float32),
            pltpu.VMEM((((RPT + L - 1) // L) * L,), jnp.float32),
            pltpu.VMEM_SHARED((SLOTS, D), jnp.float32),
            pltpu.VMEM_SHARED((SLOTS, L), jnp.float32),
            pltpu.SemaphoreType.DMA,
            pltpu.SemaphoreType.DMA,
            pltpu.SemaphoreType.DMA,
            pltpu.SemaphoreType.DMA,
        ],
    )
    def k(nodes_h, src_h, dst_h, feat_h, acc0_h, acc1_h, cnt0_h, cnt1_h, g_h,
          self_h, tbl, nod, srcv, dstv, lsrc, lslot, gbuf, ones, gch, sbuf,
          zr, zc, cbandv, cband1, acc, cnt, sem, sg0, sg1, sg2):
        c = lax.axis_index("c")
        s = lax.axis_index("s")
        wid = s * NC + c

        pltpu.sync_copy(nodes_h, nod)

        neg1 = jnp.full((L,), -1, jnp.int32)
        def init_t(i, _):
            tbl[pl.ds(i * L, L)] = neg1
            return 0
        lax.fori_loop(0, NPAD // L, init_t, 0, unroll=8)
        iota = lax.iota(jnp.int32, L)
        def fill_t(i, _):
            plsc.store_scatter(tbl, [nod[pl.ds(i * L, L)]], iota + i * L)
            return 0
        lax.fori_loop(0, B // L, fill_t, 0, unroll=8)

        eb = wid * E_PER
        pltpu.sync_copy(src_h.at[pl.ds(eb, E_PER)], srcv)
        pltpu.sync_copy(dst_h.at[pl.ds(eb, E_PER)], dstv)

        def filt(j, m):
            d = dstv[pl.ds(j * L, L)]
            sl = plsc.load_gather(tbl, [d])
            msk = sl >= 0
            cum = plsc.cumsum(msk.astype(jnp.int32))
            pos = m + cum - 1
            plsc.store_scatter(lsrc, [pos], srcv[pl.ds(j * L, L)], mask=msk)
            plsc.store_scatter(lslot, [pos], sl, mask=msk)
            return m + jnp.max(cum)
        m = lax.fori_loop(0, E_PER // L, filt, jnp.int32(0), unroll=8)

        zero_i = jnp.zeros((L,), jnp.int32)
        dummy = jnp.full((L,), B, jnp.int32)
        for p in range(K // L):
            pidx = m + iota + p * L
            plsc.store_scatter(lsrc, [pidx], zero_i)
            plsc.store_scatter(lslot, [pidx], dummy)

        nch = (m + (K - 1)) // K
        sgs = [sg0, sg1, sg2]
        NB = 3
        def issue(o, b):
            pltpu.async_copy(feat_h.at[lsrc.at[pl.ds(o, K)]],
                             gbuf.at[b], sgs[b])

        def drain(o, b):
            pltpu.make_async_copy(feat_h.at[lsrc.at[pl.ds(o, K)]],
                                  gbuf.at[b], sgs[b]).wait()

        for b in range(NB):
            @pl.when(b < nch)
            def _(b=b):
                issue(b * K, b)

        bb = wid * B_PER
        def fill_g(i, _):
            gch[pl.ds(i * L, L)] = plsc.load_gather(tbl, [nod[pl.ds(bb + i * L, L)]])
            return 0
        lax.fori_loop(0, B_PER // L, fill_g, 0)
        pltpu.sync_copy(gch, g_h.at[pl.ds(bb, B_PER)])
        pltpu.async_copy(feat_h.at[nod.at[pl.ds(bb, B_PER)]], sbuf, sem).wait()
        pltpu.sync_copy(sbuf, self_h.at[pl.ds(bb, B_PER)])

        zero = jnp.zeros((L,), jnp.float32)
        one = jnp.ones((L,), jnp.float32)
        def zfill(i, _):
            for q in range(D // L):
                zr[i, pl.ds(q * L, L)] = zero
            zc[i] = zero
            return 0
        lax.fori_loop(0, RPT, zfill, 0, unroll=4)
        def ofill(i, _):
            ones[i] = one
            return 0
        lax.fori_loop(0, K, ofill, 0, unroll=8)
        r0 = s * RPT
        pltpu.sync_copy(zr, acc.at[pl.ds(r0, RPT)])
        pltpu.sync_copy(zc, cnt.at[pl.ds(r0, RPT)])
        plsc.subcore_barrier()

        def ring(tt, _):
            for b in range(NB):
                t = tt * NB + b
                @pl.when(t < nch)
                def _(b=b, t=t):
                    o = pl.multiple_of(t * K, K)
                    drain(o, b)
                    idx = lslot.at[pl.ds(o, K)]
                    pltpu.sync_copy(gbuf.at[b], acc.at[idx], add=True)
                    pltpu.sync_copy(ones, cnt.at[idx], add=True)
                    tn = t + NB
                    @pl.when(tn < nch)
                    def _():
                        issue(pl.multiple_of(tn * K, K), b)
            return 0
        lax.fori_loop(0, (nch + NB - 1) // NB, ring, 0)
        plsc.subcore_barrier()

        pltpu.sync_copy(cnt.at[pl.ds(r0, RPT)], cbandv.at[pl.ds(0, RPT)])
        zidx = jnp.zeros((L,), jnp.int32)
        for t in range((RPT + L - 1) // L):
            rows = iota + t * L
            cband1[pl.ds(t * L, L)] = plsc.load_gather(cbandv, [rows, zidx])

        @pl.when(c == 0)
        def _():
            pltpu.sync_copy(acc.at[pl.ds(r0, RPT)], acc0_h.at[pl.ds(r0, RPT)])
            pltpu.sync_copy(cband1.at[pl.ds(0, RPT)], cnt0_h.at[pl.ds(r0, RPT)])
        @pl.when(c == 1)
        def _():
            pltpu.sync_copy(acc.at[pl.ds(r0, RPT)], acc1_h.at[pl.ds(r0, RPT)])
            pltpu.sync_copy(cband1.at[pl.ds(0, RPT)], cnt1_h.at[pl.ds(r0, RPT)])

    return k(nodes, src, dst, features)


def _phase_c(selfb, g, acc0, acc1, cnt0, cnt1, W_enc, weight):
    B, D = selfb.shape
    SLOTS = acc0.shape[0]

    def body(sref, gref, a0, a1, c0, c1, we, wc, ob):
        gv = gref[...]
        onehot = (gv[:, None] ==
                  lax.broadcasted_iota(jnp.int32, (B, SLOTS), 1)
                  ).astype(jnp.float32)
        macc = jnp.concatenate(
            [a0[...] + a1[...], (c0[...] + c1[...])[:, None]], axis=1)
        p = lax.dot_general(onehot, macc, (((1,), (0,)), ((), ())),
                            preferred_element_type=jnp.float32)
        neigh = p[:, :D] / jnp.maximum(p[:, D:D + 1], 1.0)
        w1 = we[:, :D]
        w2 = we[:, D:]
        e = lax.dot_general(sref[...], w1, (((1,), (1,)), ((), ())),
                            preferred_element_type=jnp.float32)
        e = e + lax.dot_general(neigh, w2, (((1,), (1,)), ((), ())),
                                preferred_element_type=jnp.float32)
        e = jnp.maximum(e, 0.0)
        ob[...] = lax.dot_general(e, wc[...], (((1,), (1,)), ((), ())),
                                  preferred_element_type=jnp.float32)

    return pl.pallas_call(
        body,
        out_shape=jax.ShapeDtypeStruct((B, weight.shape[0]), jnp.float32),
    )(selfb, g, acc0, acc1, cnt0, cnt1, W_enc, weight)


def kernel(nodes, edge_index, features, W_enc, weight):
    src = edge_index[0]
    dst = edge_index[1]
    acc0, acc1, cnt0, cnt1, g, selfb = _phase_a(nodes, src, dst, features)
    return _phase_c(selfb, g, acc0, acc1, cnt0, cnt1, W_enc, weight)
